# baseline (speedup 1.0000x reference)
import functools

import jax
import jax.numpy as jnp
from jax import lax
from jax.experimental import pallas as pl
from jax.experimental.pallas import tpu as pltpu
from jax.experimental.pallas import tpu_sc as plsc

_STRIPE = 1024


def _build_gather(B, V, D):
  info = plsc.get_sparse_core_info()
  num_workers = info.num_cores * info.num_subcores
  n_stripes = (V + _STRIPE - 1) // _STRIPE
  rounds = (n_stripes + num_workers - 1) // num_workers
  max_start = (V - _STRIPE) & ~127
  tail_lo = max_start + _STRIPE

  mesh = plsc.VectorSubcoreMesh(core_axis_name="c", subcore_axis_name="s")

  @functools.partial(
      pl.kernel,
      mesh=mesh,
      out_type=jax.ShapeDtypeStruct((B, D), jnp.float32),
      compiler_params=pltpu.CompilerParams(
          skip_device_barrier=True, needs_layout_passes=False),
      scratch_types=[
          pltpu.VMEM((B,), jnp.int32),
          pltpu.VMEM((D, _STRIPE), jnp.float32),
          pltpu.VMEM((B + 16,), jnp.int32),
          pltpu.VMEM((B + 16,), jnp.int32),
          pltpu.VMEM((16, D), jnp.float32),
          pltpu.SemaphoreType.DMA,
          pltpu.SemaphoreType.DMA,
      ],
  )
  def gather_kernel(idx_hbm, table_hbm, tail_hbm, out_hbm, idx_v, stripe_v,
                    mv, mb, rows_v, sem_in, sem_out):
    wid = lax.axis_index("s") * info.num_cores + lax.axis_index("c")
    pltpu.sync_copy(idx_hbm, idx_v)
    iota = lax.iota(jnp.int32, 16)

    def run_round(r, carry):
      s_id = r * num_workers + wid
      start = pl.multiple_of(jnp.minimum(s_id * _STRIPE, max_start), 128)
      stripe_in = pltpu.make_async_copy(
          table_hbm.at[:, pl.ds(start, _STRIPE)], stripe_v, sem_in)
      stripe_in.start()

      total = jnp.int32(0)
      for g in range(B // 16):
        vec = idx_v[pl.ds(g * 16, 16)]
        mask = (vec >> 10) == s_id
        pos = total + plsc.cumsum(mask.astype(jnp.int32)) - 1
        plsc.store_scatter(mv, [pos], vec, mask=mask)
        plsc.store_scatter(mb, [pos], iota + g * 16, mask=mask)
        total = pos[15] + 1

      stripe_in.wait()

      def emit_group(gi, carry2):
        base = gi * 16
        vvec = mv[pl.ds(base, 16)]
        bvec = mb[pl.ds(base, 16)]
        col_vec = jnp.clip(vvec - start, 0, _STRIPE - 1)
        copies = []
        for j in range(16):
          valid = base + j < total
          vj = vvec[j]
          colj = jnp.broadcast_to(col_vec[j], (16,))
          for q in range(D // 16):
            rows_v[j, pl.ds(q * 16, 16)] = plsc.load_gather(
                stripe_v, [iota + q * 16, colj])

          @pl.when(jnp.logical_and(valid, vj >= tail_lo))
          def _():
            pltpu.sync_copy(tail_hbm.at[vj - tail_lo], rows_v.at[j])

          copies.append(
              (valid,
               pltpu.make_async_copy(
                   rows_v.at[j], out_hbm.at[bvec[j]], sem_out))
          )
        for valid, c in copies:
          @pl.when(valid)
          def _():
            c.start()
        for valid, c in copies:
          @pl.when(valid)
          def _():
            c.wait()
        return carry2

      lax.fori_loop(0, (total + 15) // 16, emit_group, jnp.int32(0))
      return carry

    lax.fori_loop(0, rounds, run_round, jnp.int32(0))

  return gather_kernel


def kernel(inputs, table):
  B = inputs.shape[0]
  V, D = table.shape
  idx = inputs.astype(jnp.int32)
  tail_lo = ((V - _STRIPE) & ~127) + _STRIPE
  tail = lax.slice(table, (tail_lo, 0), (V, D))
  return _build_gather(B, V, D)(idx, table.T, tail)


# speedup vs baseline: 1.0567x; 1.0567x over previous
import functools

import jax
import jax.numpy as jnp
from jax import lax
from jax.experimental import pallas as pl
from jax.experimental.pallas import tpu as pltpu
from jax.experimental.pallas import tpu_sc as plsc

_STRIPE = 1536


def _build_gather(B, V, D):
  info = plsc.get_sparse_core_info()
  num_workers = info.num_cores * info.num_subcores
  n_stripes = (V + _STRIPE - 1) // _STRIPE
  rounds = (n_stripes + num_workers - 1) // num_workers
  max_start = (V - _STRIPE) & ~127
  tail_lo = max_start + _STRIPE
  b_bits = (B - 1).bit_length()

  mesh = plsc.VectorSubcoreMesh(core_axis_name="c", subcore_axis_name="s")

  @functools.partial(
      pl.kernel,
      mesh=mesh,
      out_type=jax.ShapeDtypeStruct((B, D), jnp.float32),
      compiler_params=pltpu.CompilerParams(
          skip_device_barrier=True, needs_layout_passes=False),
      scratch_types=[
          pltpu.VMEM((B,), jnp.int32),
          pltpu.VMEM((B,), jnp.int32),
          pltpu.VMEM((D, _STRIPE), jnp.float32),
          pltpu.VMEM((B + 16,), jnp.int32),
          pltpu.VMEM((16, D), jnp.float32),
          pltpu.SemaphoreType.DMA,
          pltpu.SemaphoreType.DMA,
      ],
  )
  def gather_kernel(idx_hbm, table_hbm, tail_hbm, out_hbm, ov_v, pv_v,
                    stripe_v, ml, rows_v, sem_in, sem_out):
    wid = lax.axis_index("s") * info.num_cores + lax.axis_index("c")
    pltpu.sync_copy(idx_hbm, pv_v)
    iota = lax.iota(jnp.int32, 16)

    def precompute(g, carry):
      vec = pv_v[pl.ds(g * 16, 16)]
      ov_v[pl.ds(g * 16, 16)] = vec // _STRIPE
      pv_v[pl.ds(g * 16, 16)] = (vec << b_bits) | (iota + g * 16)
      return carry

    lax.fori_loop(0, B // 16, precompute, jnp.int32(0), unroll=8)

    def run_round(r, carry):
      s_id = r * num_workers + wid

      @pl.when(s_id < n_stripes)
      def _():
        start = pl.multiple_of(jnp.minimum(s_id * _STRIPE, max_start), 128)
        stripe_in = pltpu.make_async_copy(
            table_hbm.at[:, pl.ds(start, _STRIPE)], stripe_v, sem_in)
        stripe_in.start()

        def scan_vec(g, cur):
          mask = ov_v[pl.ds(g * 16, 16)] == s_id
          cnt = plsc.all_reduce_population_count(mask)[0]

          @pl.when(cnt > 0)
          def _():
            pos = cur + plsc.cumsum(mask.astype(jnp.int32)) - 1
            plsc.store_scatter(
                ml, [pos], pv_v[pl.ds(g * 16, 16)], mask=mask)

          return cur + cnt

        total = lax.fori_loop(0, B // 16, scan_vec, jnp.int32(0), unroll=8)

        stripe_in.wait()

        def emit_group(gi, carry2):
          base = gi * 16
          pvec = ml[pl.ds(base, 16)]
          vvec = pvec >> b_bits
          bvec = pvec & (B - 1)
          col_vec = jnp.clip(vvec - start, 0, _STRIPE - 1)
          copies = []
          for j in range(16):
            valid = base + j < total
            vj = vvec[j]
            colj = jnp.broadcast_to(col_vec[j], (16,))
            for q in range(D // 16):
              rows_v[j, pl.ds(q * 16, 16)] = plsc.load_gather(
                  stripe_v, [iota + q * 16, colj])

            @pl.when(jnp.logical_and(valid, vj >= tail_lo))
            def _():
              pltpu.sync_copy(tail_hbm.at[vj - tail_lo], rows_v.at[j])

            copies.append(
                (valid,
                 pltpu.make_async_copy(
                     rows_v.at[j], out_hbm.at[bvec[j]], sem_out))
            )
          for valid, c in copies:
            @pl.when(valid)
            def _():
              c.start()
          for valid, c in copies:
            @pl.when(valid)
            def _():
              c.wait()
          return carry2

        lax.fori_loop(0, (total + 15) // 16, emit_group, jnp.int32(0))

      return carry

    lax.fori_loop(0, rounds, run_round, jnp.int32(0))

  return gather_kernel


def kernel(inputs, table):
  B = inputs.shape[0]
  V, D = table.shape
  idx = inputs.astype(jnp.int32)
  tail_lo = ((V - _STRIPE) & ~127) + _STRIPE
  tail = lax.slice(table, (tail_lo, 0), (V, D))
  return _build_gather(B, V, D)(idx, table.T, tail)


# speedup vs baseline: 1.1195x; 1.0594x over previous
import functools

import jax
import jax.numpy as jnp
from jax import lax
from jax.experimental import pallas as pl
from jax.experimental.pallas import tpu as pltpu
from jax.experimental.pallas import tpu_sc as plsc

_STRIPE = 768


def _build_gather(B, V, D):
  info = plsc.get_sparse_core_info()
  num_workers = info.num_cores * info.num_subcores
  n_stripes = (V + _STRIPE - 1) // _STRIPE
  rounds = (n_stripes + num_workers - 1) // num_workers
  max_start = (V - _STRIPE) & ~127
  tail_lo = max_start + _STRIPE
  b_bits = (B - 1).bit_length()

  mesh = plsc.VectorSubcoreMesh(core_axis_name="c", subcore_axis_name="s")

  @functools.partial(
      pl.kernel,
      mesh=mesh,
      out_type=jax.ShapeDtypeStruct((B, D), jnp.float32),
      compiler_params=pltpu.CompilerParams(
          skip_device_barrier=True, needs_layout_passes=False),
      scratch_types=[
          pltpu.VMEM((B,), jnp.int32),
          pltpu.VMEM((B + 16,), jnp.int32),
          pltpu.VMEM((B + 16,), jnp.int32),
          pltpu.VMEM((D, _STRIPE), jnp.float32),
          pltpu.VMEM((D, _STRIPE), jnp.float32),
          pltpu.VMEM((16, D), jnp.float32),
          pltpu.SemaphoreType.DMA,
          pltpu.SemaphoreType.DMA,
          pltpu.SemaphoreType.DMA,
      ],
  )
  def gather_kernel(idx_hbm, table_hbm, tail_hbm, out_hbm, idx_v, ml, mr,
                    stripe_a, stripe_b, rows_v, sem_a, sem_b, sem_out):
    wid = lax.axis_index("s") * info.num_cores + lax.axis_index("c")
    pltpu.sync_copy(idx_hbm, idx_v)
    iota = lax.iota(jnp.int32, 16)
    bufs = [(stripe_a, sem_a), (stripe_b, sem_b)]

    def stripe_copy(r, buf, sem):
      s_id = r * num_workers + wid
      start = pl.multiple_of(jnp.minimum(s_id * _STRIPE, max_start), 128)
      return pltpu.make_async_copy(
          table_hbm.at[:, pl.ds(start, _STRIPE)], buf, sem)

    stripe_copy(0, *bufs[0]).start()

    def bucket_vec(g, cur):
      vec = idx_v[pl.ds(g * 16, 16)]
      mask = (vec // _STRIPE) % num_workers == wid
      cnt = plsc.all_reduce_population_count(mask)[0]

      @pl.when(cnt > 0)
      def _():
        pos = cur + plsc.cumsum(mask.astype(jnp.int32)) - 1
        plsc.store_scatter(
            ml, [pos], (vec << b_bits) | (iota + g * 16), mask=mask)

      return cur + cnt

    n_own = lax.fori_loop(0, B // 16, bucket_vec, jnp.int32(0), unroll=8)

    for r in range(rounds):
      s_id = r * num_workers + wid
      active = s_id < n_stripes
      buf, sem = bufs[r % 2]

      @pl.when(active)
      def _():
        start = pl.multiple_of(jnp.minimum(s_id * _STRIPE, max_start), 128)

        def select_vec(g, cur):
          pvec = ml[pl.ds(g * 16, 16)]
          vvec = pvec >> b_bits
          lane_ok = g * 16 + iota < n_own
          mask = jnp.logical_and(vvec // _STRIPE == s_id, lane_ok)
          cnt = plsc.all_reduce_population_count(mask)[0]

          @pl.when(cnt > 0)
          def _():
            pos = cur + plsc.cumsum(mask.astype(jnp.int32)) - 1
            plsc.store_scatter(mr, [pos], pvec, mask=mask)

          return cur + cnt

        total = lax.fori_loop(
            0, (n_own + 15) // 16, select_vec, jnp.int32(0))

        stripe_copy(r, buf, sem).wait()
        if r + 1 < rounds:
          nxt = (r + 1) * num_workers + wid

          @pl.when(nxt < n_stripes)
          def _():
            stripe_copy(r + 1, *bufs[(r + 1) % 2]).start()

        def emit_group(gi, carry2):
          base = gi * 16
          pvec = mr[pl.ds(base, 16)]
          vvec = pvec >> b_bits
          bvec = pvec & (B - 1)
          col_vec = jnp.clip(vvec - start, 0, _STRIPE - 1)
          copies = []
          for j in range(16):
            valid = base + j < total
            vj = vvec[j]
            colj = jnp.broadcast_to(col_vec[j], (16,))
            for q in range(D // 16):
              rows_v[j, pl.ds(q * 16, 16)] = plsc.load_gather(
                  buf, [iota + q * 16, colj])

            @pl.when(jnp.logical_and(valid, vj >= tail_lo))
            def _():
              pltpu.sync_copy(tail_hbm.at[vj - tail_lo], rows_v.at[j])

            copies.append(
                (valid,
                 pltpu.make_async_copy(
                     rows_v.at[j], out_hbm.at[bvec[j]], sem_out))
            )
          for valid, c in copies:
            @pl.when(valid)
            def _():
              c.start()
          for valid, c in copies:
            @pl.when(valid)
            def _():
              c.wait()
          return carry2

        lax.fori_loop(0, (total + 15) // 16, emit_group, jnp.int32(0))

  return gather_kernel


def kernel(inputs, table):
  B = inputs.shape[0]
  V, D = table.shape
  idx = inputs.astype(jnp.int32)
  tail_lo = ((V - _STRIPE) & ~127) + _STRIPE
  tail = lax.slice(table, (tail_lo, 0), (V, D))
  return _build_gather(B, V, D)(idx, table.T, tail)


# speedup vs baseline: 1.1562x; 1.0328x over previous
import functools

import jax
import jax.numpy as jnp
from jax import lax
from jax.experimental import pallas as pl
from jax.experimental.pallas import tpu as pltpu
from jax.experimental.pallas import tpu_sc as plsc

_STRIPE = 768


def _build_gather(B, V, D):
  info = plsc.get_sparse_core_info()
  num_workers = info.num_cores * info.num_subcores
  n_stripes = (V + _STRIPE - 1) // _STRIPE
  rounds = (n_stripes + num_workers - 1) // num_workers
  max_start = (V - _STRIPE) & ~127
  tail_lo = max_start + _STRIPE
  b_bits = (B - 1).bit_length()

  mesh = plsc.VectorSubcoreMesh(core_axis_name="c", subcore_axis_name="s")

  @functools.partial(
      pl.kernel,
      mesh=mesh,
      out_type=jax.ShapeDtypeStruct((B, D), jnp.float32),
      compiler_params=pltpu.CompilerParams(
          skip_device_barrier=True, needs_layout_passes=False),
      scratch_types=[
          pltpu.VMEM((B,), jnp.int32),
          pltpu.VMEM((B + 16,), jnp.int32),
          pltpu.VMEM((B + 16,), jnp.int32),
          pltpu.VMEM((D, _STRIPE), jnp.float32),
          pltpu.VMEM((D, _STRIPE), jnp.float32),
          pltpu.VMEM((16, D), jnp.float32),
          pltpu.SemaphoreType.DMA,
          pltpu.SemaphoreType.DMA,
          pltpu.SemaphoreType.DMA,
      ],
  )
  def gather_kernel(idx_hbm, table_hbm, tail_hbm, out_hbm, idx_v, ml, mr,
                    stripe_a, stripe_b, rows_v, sem_a, sem_b, sem_out):
    wid = lax.axis_index("s") * info.num_cores + lax.axis_index("c")
    pltpu.sync_copy(idx_hbm, idx_v)
    iota = lax.iota(jnp.int32, 16)
    bufs = [(stripe_a, sem_a), (stripe_b, sem_b)]

    def stripe_copy(r, buf, sem):
      s_id = r * num_workers + wid
      start = pl.multiple_of(jnp.minimum(s_id * _STRIPE, max_start), 128)
      return pltpu.make_async_copy(
          table_hbm.at[:, pl.ds(start, _STRIPE)], buf, sem)

    stripe_copy(0, *bufs[0]).start()

    lane15 = jnp.full((16,), 15, jnp.int32)

    def bucket_vec(g, cur_vec):
      vec = idx_v[pl.ds(g * 16, 16)]
      mask = (vec // _STRIPE) % num_workers == wid
      pos = cur_vec + plsc.cumsum(mask.astype(jnp.int32)) - 1
      plsc.store_scatter(
          ml, [pos], (vec << b_bits) | (iota + g * 16), mask=mask)
      return pos[lane15] + 1

    n_own = lax.fori_loop(
        0, B // 16, bucket_vec, jnp.zeros((16,), jnp.int32), unroll=8)[0]

    for r in range(rounds):
      s_id = r * num_workers + wid
      active = s_id < n_stripes
      buf, sem = bufs[r % 2]

      @pl.when(active)
      def _():
        start = pl.multiple_of(jnp.minimum(s_id * _STRIPE, max_start), 128)

        def select_vec(g, cur):
          pvec = ml[pl.ds(g * 16, 16)]
          vvec = pvec >> b_bits
          lane_ok = g * 16 + iota < n_own
          mask = jnp.logical_and(vvec // _STRIPE == s_id, lane_ok)
          cnt = plsc.all_reduce_population_count(mask)[0]

          @pl.when(cnt > 0)
          def _():
            pos = cur + plsc.cumsum(mask.astype(jnp.int32)) - 1
            plsc.store_scatter(mr, [pos], pvec, mask=mask)

          return cur + cnt

        total = lax.fori_loop(
            0, (n_own + 15) // 16, select_vec, jnp.int32(0))

        stripe_copy(r, buf, sem).wait()
        if r + 1 < rounds:
          nxt = (r + 1) * num_workers + wid

          @pl.when(nxt < n_stripes)
          def _():
            stripe_copy(r + 1, *bufs[(r + 1) % 2]).start()

        def emit_group(gi, carry2):
          base = gi * 16
          pvec = mr[pl.ds(base, 16)]
          vvec = pvec >> b_bits
          bvec = pvec & (B - 1)
          col_vec = jnp.clip(vvec - start, 0, _STRIPE - 1)
          for d0 in range(D):
            plsc.store_scatter(
                rows_v,
                [iota, jnp.full((16,), d0, jnp.int32)],
                plsc.load_gather(buf, [jnp.full((16,), d0, jnp.int32),
                                       col_vec]),
            )
          tail_cnt = plsc.all_reduce_population_count(vvec >= tail_lo)[0]

          @pl.when(tail_cnt > 0)
          def _():
            for j in range(16):
              vj = vvec[j]

              @pl.when(jnp.logical_and(base + j < total, vj >= tail_lo))
              def _():
                pltpu.sync_copy(tail_hbm.at[vj - tail_lo], rows_v.at[j])

          copies = []
          for j in range(16):
            valid = base + j < total
            copies.append(
                (valid,
                 pltpu.make_async_copy(
                     rows_v.at[j], out_hbm.at[bvec[j]], sem_out))
            )
          for valid, c in copies:
            @pl.when(valid)
            def _():
              c.start()
          for valid, c in copies:
            @pl.when(valid)
            def _():
              c.wait()
          return carry2

        lax.fori_loop(0, (total + 15) // 16, emit_group, jnp.int32(0))

  return gather_kernel


def kernel(inputs, table):
  B = inputs.shape[0]
  V, D = table.shape
  idx = inputs.astype(jnp.int32)
  tail_lo = ((V - _STRIPE) & ~127) + _STRIPE
  tail = lax.slice(table, (tail_lo, 0), (V, D))
  return _build_gather(B, V, D)(idx, table.T, tail)


# speedup vs baseline: 1.1667x; 1.0091x over previous
import functools

import jax
import jax.numpy as jnp
from jax import lax
from jax.experimental import pallas as pl
from jax.experimental.pallas import tpu as pltpu
from jax.experimental.pallas import tpu_sc as plsc

_STRIPE = 768


def _build_gather(B, V, D):
  info = plsc.get_sparse_core_info()
  num_workers = info.num_cores * info.num_subcores
  n_stripes = (V + _STRIPE - 1) // _STRIPE
  rounds = (n_stripes + num_workers - 1) // num_workers
  max_start = (V - _STRIPE) & ~127
  tail_lo = max_start + _STRIPE
  b_bits = (B - 1).bit_length()

  mesh = plsc.VectorSubcoreMesh(core_axis_name="c", subcore_axis_name="s")

  @functools.partial(
      pl.kernel,
      mesh=mesh,
      out_type=jax.ShapeDtypeStruct((B, D), jnp.float32),
      compiler_params=pltpu.CompilerParams(
          skip_device_barrier=True, needs_layout_passes=False),
      scratch_types=[
          pltpu.VMEM((B,), jnp.int32),
          pltpu.VMEM((B + 16,), jnp.int32),
          pltpu.VMEM((B + 16,), jnp.int32),
          pltpu.VMEM((D, _STRIPE), jnp.float32),
          pltpu.VMEM((D, _STRIPE), jnp.float32),
          pltpu.VMEM((16, D), jnp.float32),
          pltpu.SemaphoreType.DMA,
          pltpu.SemaphoreType.DMA,
          pltpu.SemaphoreType.DMA,
      ],
  )
  def gather_kernel(idx_hbm, table_hbm, tail_hbm, out_hbm, idx_v, ml, mr,
                    stripe_a, stripe_b, rows_v, sem_a, sem_b, sem_out):
    wid = lax.axis_index("s") * info.num_cores + lax.axis_index("c")
    pltpu.sync_copy(idx_hbm, idx_v)
    iota = lax.iota(jnp.int32, 16)
    bufs = [(stripe_a, sem_a), (stripe_b, sem_b)]

    def stripe_copy(r, buf, sem):
      s_id = r * num_workers + wid
      start = pl.multiple_of(jnp.minimum(s_id * _STRIPE, max_start), 128)
      return pltpu.make_async_copy(
          table_hbm.at[:, pl.ds(start, _STRIPE)], buf, sem)

    stripe_copy(0, *bufs[0]).start()

    lane15 = jnp.full((16,), 15, jnp.int32)

    def bucket_vec(g, cur_vec):
      vec = idx_v[pl.ds(g * 16, 16)]
      mask = (vec // _STRIPE) % num_workers == wid
      pos = cur_vec + plsc.cumsum(mask.astype(jnp.int32)) - 1
      plsc.store_scatter(
          ml, [pos], (vec << b_bits) | (iota + g * 16), mask=mask)
      return pos[lane15] + 1

    n_own = lax.fori_loop(
        0, B // 16, bucket_vec, jnp.zeros((16,), jnp.int32), unroll=8)[0]

    for r in range(rounds):
      s_id = r * num_workers + wid
      active = s_id < n_stripes
      buf, sem = bufs[r % 2]

      @pl.when(active)
      def _():
        start = pl.multiple_of(jnp.minimum(s_id * _STRIPE, max_start), 128)

        if r + 1 < rounds:
          nxt = (r + 1) * num_workers + wid

          @pl.when(nxt < n_stripes)
          def _():
            stripe_copy(r + 1, *bufs[(r + 1) % 2]).start()

        def select_vec(g, cur):
          pvec = ml[pl.ds(g * 16, 16)]
          vvec = pvec >> b_bits
          lane_ok = g * 16 + iota < n_own
          mask = jnp.logical_and(vvec // _STRIPE == s_id, lane_ok)
          cnt = plsc.all_reduce_population_count(mask)[0]

          @pl.when(cnt > 0)
          def _():
            pos = cur + plsc.cumsum(mask.astype(jnp.int32)) - 1
            plsc.store_scatter(mr, [pos], pvec, mask=mask)

          return cur + cnt

        total = lax.fori_loop(
            0, (n_own + 15) // 16, select_vec, jnp.int32(0))

        stripe_copy(r, buf, sem).wait()

        def emit_group(gi, carry2):
          base = gi * 16
          pvec = mr[pl.ds(base, 16)]
          vvec = pvec >> b_bits
          bvec = pvec & (B - 1)
          col_vec = jnp.clip(vvec - start, 0, _STRIPE - 1)
          for d0 in range(D):
            plsc.store_scatter(
                rows_v,
                [iota, jnp.full((16,), d0, jnp.int32)],
                plsc.load_gather(buf, [jnp.full((16,), d0, jnp.int32),
                                       col_vec]),
            )
          tail_cnt = plsc.all_reduce_population_count(vvec >= tail_lo)[0]

          @pl.when(tail_cnt > 0)
          def _():
            for j in range(16):
              vj = vvec[j]

              @pl.when(jnp.logical_and(base + j < total, vj >= tail_lo))
              def _():
                pltpu.sync_copy(tail_hbm.at[vj - tail_lo], rows_v.at[j])

          copies = []
          for j in range(16):
            valid = base + j < total
            copies.append(
                (valid,
                 pltpu.make_async_copy(
                     rows_v.at[j], out_hbm.at[bvec[j]], sem_out))
            )
          for valid, c in copies:
            @pl.when(valid)
            def _():
              c.start()
          for valid, c in copies:
            @pl.when(valid)
            def _():
              c.wait()
          return carry2

        lax.fori_loop(0, (total + 15) // 16, emit_group, jnp.int32(0))

  return gather_kernel


def kernel(inputs, table):
  B = inputs.shape[0]
  V, D = table.shape
  idx = inputs.astype(jnp.int32)
  tail_lo = ((V - _STRIPE) & ~127) + _STRIPE
  tail = lax.slice(table, (tail_lo, 0), (V, D))
  return _build_gather(B, V, D)(idx, table.T, tail)


# speedup vs baseline: 1.3630x; 1.1683x over previous
import functools

import jax
import jax.numpy as jnp
from jax import lax
from jax.experimental import pallas as pl
from jax.experimental.pallas import tpu as pltpu
from jax.experimental.pallas import tpu_sc as plsc

_STRIPE = 512


def _build_gather(B, V, D):
  info = plsc.get_sparse_core_info()
  num_workers = info.num_cores * info.num_subcores
  n_stripes = (V + _STRIPE - 1) // _STRIPE
  rounds = (n_stripes + num_workers - 1) // num_workers
  max_start = (V - _STRIPE) & ~127
  tail_lo = max_start + _STRIPE
  b_bits = (B - 1).bit_length()

  mesh = plsc.VectorSubcoreMesh(core_axis_name="c", subcore_axis_name="s")

  @functools.partial(
      pl.kernel,
      mesh=mesh,
      out_type=jax.ShapeDtypeStruct((B, D), jnp.float32),
      compiler_params=pltpu.CompilerParams(
          skip_device_barrier=True, needs_layout_passes=False),
      scratch_types=[
          pltpu.VMEM((B,), jnp.int32),
          pltpu.VMEM((B + 16,), jnp.int32),
          pltpu.VMEM((B + 16,), jnp.int32),
          pltpu.VMEM((D, _STRIPE), jnp.float32),
          pltpu.VMEM((D, _STRIPE), jnp.float32),
          pltpu.VMEM((D, _STRIPE), jnp.float32),
          pltpu.VMEM((16, D), jnp.float32),
          pltpu.SemaphoreType.DMA,
          pltpu.SemaphoreType.DMA,
          pltpu.SemaphoreType.DMA,
          pltpu.SemaphoreType.DMA,
      ],
  )
  def gather_kernel(idx_hbm, table_hbm, tail_hbm, out_hbm, idx_v, ml, mr,
                    stripe_a, stripe_b, stripe_c, rows_v, sem_a, sem_b,
                    sem_c, sem_out):
    wid = lax.axis_index("s") * info.num_cores + lax.axis_index("c")
    pltpu.sync_copy(idx_hbm, idx_v)
    iota = lax.iota(jnp.int32, 16)
    bufs = [(stripe_a, sem_a), (stripe_b, sem_b), (stripe_c, sem_c)]

    def stripe_copy(r, buf, sem):
      s_id = r * num_workers + wid
      start = pl.multiple_of(jnp.minimum(s_id * _STRIPE, max_start), 128)
      return pltpu.make_async_copy(
          table_hbm.at[:, pl.ds(start, _STRIPE)], buf, sem)

    stripe_copy(0, *bufs[0]).start()
    if rounds > 1:
      stripe_copy(1, *bufs[1]).start()

    lane15 = jnp.full((16,), 15, jnp.int32)

    def bucket_vec(g, cur_vec):
      vec = idx_v[pl.ds(g * 16, 16)]
      mask = (vec // _STRIPE) % num_workers == wid
      pos = cur_vec + plsc.cumsum(mask.astype(jnp.int32)) - 1
      plsc.store_scatter(
          ml, [pos], (vec << b_bits) | (iota + g * 16), mask=mask)
      return pos[lane15] + 1

    n_own = lax.fori_loop(
        0, B // 16, bucket_vec, jnp.zeros((16,), jnp.int32), unroll=8)[0]

    for r in range(rounds):
      s_id = r * num_workers + wid
      active = s_id < n_stripes
      buf, sem = bufs[r % 3]

      @pl.when(active)
      def _():
        start = pl.multiple_of(jnp.minimum(s_id * _STRIPE, max_start), 128)

        if r + 2 < rounds:
          nxt = (r + 2) * num_workers + wid

          @pl.when(nxt < n_stripes)
          def _():
            stripe_copy(r + 2, *bufs[(r + 2) % 3]).start()

        def select_vec(g, cur):
          pvec = ml[pl.ds(g * 16, 16)]
          vvec = pvec >> b_bits
          lane_ok = g * 16 + iota < n_own
          mask = jnp.logical_and(vvec // _STRIPE == s_id, lane_ok)
          cnt = plsc.all_reduce_population_count(mask)[0]

          @pl.when(cnt > 0)
          def _():
            pos = cur + plsc.cumsum(mask.astype(jnp.int32)) - 1
            plsc.store_scatter(mr, [pos], pvec, mask=mask)

          return cur + cnt

        total = lax.fori_loop(
            0, (n_own + 15) // 16, select_vec, jnp.int32(0))

        stripe_copy(r, buf, sem).wait()

        def emit_group(gi, carry2):
          base = gi * 16
          pvec = mr[pl.ds(base, 16)]
          vvec = pvec >> b_bits
          bvec = pvec & (B - 1)
          col_vec = jnp.clip(vvec - start, 0, _STRIPE - 1)
          for d0 in range(D):
            plsc.store_scatter(
                rows_v,
                [iota, jnp.full((16,), d0, jnp.int32)],
                plsc.load_gather(buf, [jnp.full((16,), d0, jnp.int32),
                                       col_vec]),
            )
          tail_cnt = plsc.all_reduce_population_count(vvec >= tail_lo)[0]

          @pl.when(tail_cnt > 0)
          def _():
            for j in range(16):
              vj = vvec[j]

              @pl.when(jnp.logical_and(base + j < total, vj >= tail_lo))
              def _():
                pltpu.sync_copy(tail_hbm.at[vj - tail_lo], rows_v.at[j])

          copies = []
          for j in range(16):
            valid = base + j < total
            copies.append(
                (valid,
                 pltpu.make_async_copy(
                     rows_v.at[j], out_hbm.at[bvec[j]], sem_out))
            )
          for valid, c in copies:
            @pl.when(valid)
            def _():
              c.start()
          for valid, c in copies:
            @pl.when(valid)
            def _():
              c.wait()
          return carry2

        lax.fori_loop(0, (total + 15) // 16, emit_group, jnp.int32(0))

  return gather_kernel


def kernel(inputs, table):
  B = inputs.shape[0]
  V, D = table.shape
  idx = inputs.astype(jnp.int32)
  tail_lo = ((V - _STRIPE) & ~127) + _STRIPE
  tail = lax.slice(table, (tail_lo, 0), (V, D))
  return _build_gather(B, V, D)(idx, table.T, tail)
